# baseline (speedup 1.0000x reference)
import functools

import jax
import jax.numpy as jnp
from jax import lax
from jax.experimental import pallas as pl
from jax.experimental.pallas import tpu as pltpu
from jax.experimental.pallas import tpu_sc as plsc

_EPS = 1e-5
_R = 512
_K = 64


def _sc_gather(N, H, V, n_chunks):
    mesh = plsc.VectorSubcoreMesh(core_axis_name="c", subcore_axis_name="s")
    NC, NS = mesh.num_cores, mesh.num_subcores
    NW = NC * NS
    b_per_w = N // NW
    assert b_per_w == n_chunks * _K

    @functools.partial(
        pl.kernel,
        mesh=mesh,
        out_type=jax.ShapeDtypeStruct((N, H), jnp.float32),
        scratch_types=[
            pltpu.VMEM((_K,), jnp.int32),
            pltpu.VMEM((_K, H), jnp.float32),
            pltpu.SemaphoreType.DMA,
        ],
    )
    def gather_kernel(table_hbm, ids_hbm, out_hbm, idx_v, rows_v, sem):
        wid = lax.axis_index("s") * NC + lax.axis_index("c")
        base = wid * b_per_w
        for k in range(n_chunks):
            off = base + k * _K
            pltpu.sync_copy(ids_hbm.at[pl.ds(off, _K)], idx_v)
            pltpu.async_copy(table_hbm.at[idx_v], rows_v, sem).wait()
            pltpu.sync_copy(rows_v, out_hbm.at[pl.ds(off, _K)])

    return gather_kernel


def _tc_body(frame_ref, pos_ref, w_ref, b_ref, out_ref):
    emb = frame_ref[...] + pos_ref[...]
    mean = jnp.mean(emb, axis=1, keepdims=True)
    cent = emb - mean
    var = jnp.mean(cent * cent, axis=1, keepdims=True)
    normed = cent * lax.rsqrt(var + _EPS)
    out_ref[...] = normed * w_ref[...] + b_ref[...]


def kernel(frame_feat, position_ids, pos_table, ln_weight, ln_bias):
    B, S, H = frame_feat.shape
    V = pos_table.shape[0]
    N = B * S

    ids = position_ids.reshape(N).astype(jnp.int32)
    frame_r = frame_feat.reshape(N, H)
    w_r = ln_weight.reshape(1, H)
    b_r = ln_bias.reshape(1, H)

    gathered = _sc_gather(N, H, V, N // 32 // _K)(pos_table, ids)

    out = pl.pallas_call(
        _tc_body,
        grid=(N // _R,),
        in_specs=[
            pl.BlockSpec((_R, H), lambda i: (i, 0)),
            pl.BlockSpec((_R, H), lambda i: (i, 0)),
            pl.BlockSpec((1, H), lambda i: (0, 0)),
            pl.BlockSpec((1, H), lambda i: (0, 0)),
        ],
        out_specs=pl.BlockSpec((_R, H), lambda i: (i, 0)),
        out_shape=jax.ShapeDtypeStruct((N, H), jnp.float32),
    )(frame_r, gathered, w_r, b_r)
    return out.reshape(B, S, H)
